# baseline (speedup 1.0000x reference)
import jax
import jax.numpy as jnp
from jax.experimental import pallas as pl
from jax.experimental.pallas import tpu as pltpu

_TM = 512
_CORES = 2
_KCHUNKS = 4


def _mm_bias_kernel(x_ref, w_hbm, b_ref, o_ref, wf_ref, wb_ref, sems):
    j = pl.program_id(1)
    nx = wf_ref.shape[0]
    ck = nx // _KCHUNKS
    xb = x_ref[...].astype(jnp.bfloat16)

    @pl.when(j == 0)
    def _first_step():
        copies = [
            pltpu.make_async_copy(
                w_hbm.at[t * ck:(t + 1) * ck, :],
                wf_ref.at[t * ck:(t + 1) * ck, :],
                sems.at[t],
            )
            for t in range(_KCHUNKS)
        ]
        for cp in copies:
            cp.start()
        acc = jnp.zeros(o_ref.shape, jnp.float32)
        for t in range(_KCHUNKS):
            copies[t].wait()
            lo, hi = t * ck, (t + 1) * ck
            wb_ref[lo:hi, :] = wf_ref[lo:hi, :].astype(jnp.bfloat16)
            acc += jnp.dot(
                xb[:, lo:hi], wb_ref[lo:hi, :], preferred_element_type=jnp.float32
            )
        o_ref[...] = acc + b_ref[...]

    @pl.when(j > 0)
    def _steady_state():
        acc = jnp.dot(xb, wb_ref[...], preferred_element_type=jnp.float32)
        o_ref[...] = acc + b_ref[...]


def kernel(x, weight, bias):
    *lead, nx = x.shape
    nf = weight.shape[1]
    x2d = x.reshape(-1, nx)
    m = x2d.shape[0]
    inner = m // _TM // _CORES
    out = pl.pallas_call(
        _mm_bias_kernel,
        out_shape=jax.ShapeDtypeStruct((m, nf), x.dtype),
        grid=(_CORES, inner),
        in_specs=[
            pl.BlockSpec((_TM, nx), lambda c, j: (c * inner + j, 0)),
            pl.BlockSpec(memory_space=pl.ANY),
            pl.BlockSpec((1, nf), lambda c, j: (0, 0)),
        ],
        out_specs=pl.BlockSpec((_TM, nf), lambda c, j: (c * inner + j, 0)),
        scratch_shapes=[
            pltpu.VMEM((nx, nf), jnp.float32),
            pltpu.VMEM((nx, nf), jnp.bfloat16),
            pltpu.SemaphoreType.DMA((_KCHUNKS,)),
        ],
        compiler_params=pltpu.CompilerParams(
            dimension_semantics=("parallel", "arbitrary"),
            vmem_limit_bytes=56 << 20,
        ),
    )(x2d, weight, bias.reshape(1, nf))
    return out.reshape(*lead, nf)
